# baseline (speedup 1.0000x reference)
import jax
import jax.numpy as jnp
from jax import lax
from jax.experimental import pallas as pl
from jax.experimental.pallas import tpu as pltpu
from jax.experimental.pallas import tpu_sc as plsc

N = 16000
E = 256000
IN = 128
HID = 256
LAT = 64
BATCH = 16
N_PER = 1000

NUM_TILES = 16
EDGE_CHUNK = 128
CHUNKS_PER_TILE = E // (NUM_TILES * EDGE_CHUNK)
NODES_PER_TILE = N // NUM_TILES
FLUSH_CHUNK = 125
N_FLUSH = NODES_PER_TILE // FLUSH_CHUNK


def _seg_scratch():
    return [
        pltpu.VMEM((CHUNKS_PER_TILE, EDGE_CHUNK), jnp.int32),
        pltpu.VMEM((CHUNKS_PER_TILE, EDGE_CHUNK), jnp.int32),
        pltpu.VMEM((EDGE_CHUNK, 64), jnp.float32),
        pltpu.VMEM((EDGE_CHUNK, 64), jnp.float32),
        pltpu.VMEM((EDGE_CHUNK, 64), jnp.float32),
        pltpu.VMEM_SHARED((N, 64), jnp.float32),
        pltpu.SemaphoreType.DMA,
        pltpu.SemaphoreType.DMA,
    ]


def _seg_pass(c, s, tbl_pair, out_pair, zeros_h,
              idx_s, idx_d, rowsA, rowsB, rowsC, accum, gsem, ssem):
    nbase = s * NODES_PER_TILE
    pltpu.sync_copy(zeros_h, accum.at[pl.ds(nbase, NODES_PER_TILE)])
    plsc.subcore_barrier()

    t0, t1 = tbl_pair
    bufs = (rowsA, rowsB, rowsC)

    def gf(chunk, q):
        @pl.when(c == 0)
        def _():
            pltpu.async_copy(t0.at[idx_s.at[chunk]], bufs[q], gsem)

        @pl.when(c == 1)
        def _():
            pltpu.async_copy(t1.at[idx_s.at[chunk]], bufs[q], gsem)

    def gw(q):
        pltpu.make_async_copy(t0.at[idx_s.at[0]], bufs[q], gsem).wait()

    def sf(chunk, q):
        pltpu.async_copy(bufs[q], accum.at[idx_d.at[chunk]], ssem, add=True)

    def sw():
        pltpu.make_async_copy(bufs[0], accum.at[idx_d.at[0]], ssem).wait()

    gf(0, 0)
    gf(1, 1)
    gw(0); sf(0, 0); gf(2, 2)
    gw(1); sf(1, 1); sw(); gf(3, 0)
    gw(2); sf(2, 2); sw(); gf(4, 1)

    @pl.loop(0, (CHUNKS_PER_TILE - 5) // 3)
    def _(j):
        n = 3 + 3 * j
        gw(0); sf(n, 0); sw(); gf(n + 2, 2)
        gw(1); sf(n + 1, 1); sw(); gf(n + 3, 0)
        gw(2); sf(n + 2, 2); sw(); gf(n + 4, 1)

    gw(0); sf(CHUNKS_PER_TILE - 2, 0); sw()
    gw(1); sf(CHUNKS_PER_TILE - 1, 1); sw()
    sw()

    plsc.subcore_barrier()

    o0, o1 = out_pair

    @pl.when(c == 0)
    def _():
        pltpu.sync_copy(accum.at[pl.ds(nbase, NODES_PER_TILE)],
                        o0.at[pl.ds(nbase, NODES_PER_TILE)])

    @pl.when(c == 1)
    def _():
        pltpu.sync_copy(accum.at[pl.ds(nbase, NODES_PER_TILE)],
                        o1.at[pl.ds(nbase, NODES_PER_TILE)])

    plsc.subcore_barrier()


def _stage_indices(src_h, dst_h, s, idx_s, idx_d):
    pltpu.sync_copy(src_h.at[pl.ds(s * CHUNKS_PER_TILE, CHUNKS_PER_TILE)],
                    idx_s)
    pltpu.sync_copy(dst_h.at[pl.ds(s * CHUNKS_PER_TILE, CHUNKS_PER_TILE)],
                    idx_d)


def _seg_body_gc1(src_h, dst_h, zeros_h, t0, t1, out0, out1, *scr):
    c = lax.axis_index("c")
    s = lax.axis_index("s")
    _stage_indices(src_h, dst_h, s, scr[0], scr[1])
    _seg_pass(c, s, (t0, t1), (out0, out1), zeros_h, *scr)


def _seg_gc1(src2d, dst2d, zeros_h, t0, t1):
    mesh = plsc.VectorSubcoreMesh(core_axis_name="c", subcore_axis_name="s",
                                  num_cores=2, num_subcores=NUM_TILES)
    out = jax.ShapeDtypeStruct((N, 64), jnp.float32)
    f = pl.kernel(
        _seg_body_gc1,
        out_type=(out, out),
        mesh=mesh,
        compiler_params=pltpu.CompilerParams(use_tc_tiling_on_sc=False),
        scratch_types=_seg_scratch(),
    )
    return f(src2d, dst2d, zeros_h, t0, t1)


def _seg_body_gc2(src_h, dst_h, zeros_h, t0, t1, t2, t3,
                  out0, out1, out2, out3, *scr):
    c = lax.axis_index("c")
    s = lax.axis_index("s")
    _stage_indices(src_h, dst_h, s, scr[0], scr[1])
    _seg_pass(c, s, (t0, t2), (out0, out2), zeros_h, *scr)
    _seg_pass(c, s, (t1, t3), (out1, out3), zeros_h, *scr)


def _seg_gc2(src2d, dst2d, zeros_h, t0, t1, t2, t3):
    mesh = plsc.VectorSubcoreMesh(core_axis_name="c", subcore_axis_name="s",
                                  num_cores=2, num_subcores=NUM_TILES)
    out = jax.ShapeDtypeStruct((N, 64), jnp.float32)
    f = pl.kernel(
        _seg_body_gc2,
        out_type=(out, out, out, out),
        mesh=mesh,
        compiler_params=pltpu.CompilerParams(use_tc_tiling_on_sc=False),
        scratch_types=_seg_scratch(),
    )
    return f(src2d, dst2d, zeros_h, t0, t1, t2, t3)


ROWS_A = 1000


def _root_body(xb, wx, o):
    o[...] = jnp.dot(xb[...], wx[...], preferred_element_type=jnp.float32)


def _root_mm(x, W, d_in, d_out):
    return pl.pallas_call(
        _root_body,
        grid=(N // ROWS_A,),
        in_specs=[
            pl.BlockSpec((ROWS_A, d_in), lambda i: (i, 0)),
            pl.BlockSpec((d_in, d_out), lambda i: (0, 0)),
        ],
        out_specs=pl.BlockSpec((ROWS_A, d_out), lambda i: (i, 0)),
        out_shape=jax.ShapeDtypeStruct((N, d_out), jnp.float32),
    )(x, W)


def _gc1_body(a0, a1, rt, wr, b1, h1, g0, g1, g2, g3):
    agg = jnp.concatenate([a0[...], a1[...]], axis=1)
    h = jnp.dot(agg, wr[...], preferred_element_type=jnp.float32)
    h += rt[...]
    h = jnp.maximum(h + b1[...], 0.0)
    h1[...] = h
    g0[...] = h[:, 0:64]
    g1[...] = h[:, 64:128]
    g2[...] = h[:, 128:192]
    g3[...] = h[:, 192:256]


def _gc1_dense(agg0, agg1, root1, W_rel1, b1):
    g_spec = pl.BlockSpec((ROWS_A, 64), lambda i: (i, 0))
    out64 = jax.ShapeDtypeStruct((N, 64), jnp.float32)
    return pl.pallas_call(
        _gc1_body,
        grid=(N // ROWS_A,),
        in_specs=[
            g_spec,
            g_spec,
            pl.BlockSpec((ROWS_A, HID), lambda i: (i, 0)),
            pl.BlockSpec((IN, HID), lambda i: (0, 0)),
            pl.BlockSpec((1, HID), lambda i: (0, 0)),
        ],
        out_specs=[pl.BlockSpec((ROWS_A, HID), lambda i: (i, 0)),
                   g_spec, g_spec, g_spec, g_spec],
        out_shape=[jax.ShapeDtypeStruct((N, HID), jnp.float32),
                   out64, out64, out64, out64],
    )(agg0, agg1, root1, W_rel1, b1)


NL_B = 40
FLAT = N_PER * HID


def _gc2_heads_body(a0, a1, a2, a3, rt, wr, b2, wl, wls, bl, bls,
                    loc_ref, ls_ref):
    i = pl.program_id(0)
    agg = jnp.concatenate([a0[...], a1[...], a2[...], a3[...]],
                          axis=2).reshape(BATCH * NL_B, HID)
    h = jnp.dot(agg, wr[...], preferred_element_type=jnp.float32)
    h += rt[...].reshape(BATCH * NL_B, HID)
    h2 = jnp.maximum(h + b2[...], 0.0)
    flat = h2.reshape(BATCH, NL_B * HID)

    @pl.when(i == 0)
    def _():
        loc_ref[...] = jnp.broadcast_to(bl[...], (BATCH, LAT))
        ls_ref[...] = jnp.broadcast_to(bls[...], (BATCH, LAT))

    loc_ref[...] += jnp.dot(flat, wl[...].reshape(NL_B * HID, LAT),
                            preferred_element_type=jnp.float32)
    ls_ref[...] += jnp.dot(flat, wls[...].reshape(NL_B * HID, LAT),
                           preferred_element_type=jnp.float32)


def _gc2_heads(agg2s, root2, W_rel2, b2, W_loc, b_loc, W_ls, b_ls):
    a_spec = pl.BlockSpec((BATCH, NL_B, 64), lambda i: (0, i, 0))
    w_spec = pl.BlockSpec((NL_B, HID, LAT), lambda i: (i, 0, 0))
    out_spec = pl.BlockSpec((BATCH, LAT), lambda i: (0, 0))
    out = jax.ShapeDtypeStruct((BATCH, LAT), jnp.float32)
    a3d = [a.reshape(BATCH, N_PER, 64) for a in agg2s]
    return pl.pallas_call(
        _gc2_heads_body,
        grid=(N_PER // NL_B,),
        in_specs=[
            a_spec, a_spec, a_spec, a_spec,
            pl.BlockSpec((BATCH, NL_B, HID), lambda i: (0, i, 0)),
            pl.BlockSpec((HID, HID), lambda i: (0, 0)),
            pl.BlockSpec((1, HID), lambda i: (0, 0)),
            w_spec, w_spec,
            pl.BlockSpec((1, LAT), lambda i: (0, 0)),
            pl.BlockSpec((1, LAT), lambda i: (0, 0)),
        ],
        out_specs=[out_spec, out_spec],
        out_shape=[out, out],
        compiler_params=pltpu.CompilerParams(
            dimension_semantics=("arbitrary",)),
    )(*a3d, root2.reshape(BATCH, N_PER, HID), W_rel2, b2,
      W_loc.reshape(N_PER, HID, LAT), W_ls.reshape(N_PER, HID, LAT),
      b_loc, b_ls)


@jax.jit
def kernel(x, edge_index, batch, W_rel1, b1, W_root1, W_rel2, b2, W_root2,
           W_loc, b_loc, W_ls, b_ls):
    src2d = edge_index[0].reshape(E // EDGE_CHUNK, EDGE_CHUNK)
    dst2d = edge_index[1].reshape(E // EDGE_CHUNK, EDGE_CHUNK)
    zeros_h = jnp.zeros((NODES_PER_TILE, 64), jnp.float32)
    x0 = x[:, 0:64]
    x1 = x[:, 64:128]

    agg1 = _seg_gc1(src2d, dst2d, zeros_h, x0, x1)
    root1 = _root_mm(x, W_root1, IN, HID)
    h1, hg0, hg1, hg2, hg3 = _gc1_dense(
        agg1[0], agg1[1], root1, W_rel1, b1.reshape(1, HID))
    agg2s = _seg_gc2(src2d, dst2d, zeros_h, hg0, hg1, hg2, hg3)
    root2 = _root_mm(h1, W_root2, HID, HID)
    loc, ls = _gc2_heads(agg2s, root2, W_rel2, b2.reshape(1, HID),
                         W_loc, b_loc.reshape(1, LAT), W_ls,
                         b_ls.reshape(1, LAT))
    return loc, ls


# speedup vs baseline: 1.0222x; 1.0222x over previous
import jax
import jax.numpy as jnp
from jax import lax
from jax.experimental import pallas as pl
from jax.experimental.pallas import tpu as pltpu
from jax.experimental.pallas import tpu_sc as plsc

N = 16000
E = 256000
IN = 128
HID = 256
LAT = 64
BATCH = 16
N_PER = 1000

NUM_TILES = 16
EDGE_CHUNK = 128
CHUNKS_PER_TILE = E // (NUM_TILES * EDGE_CHUNK)
NODES_PER_TILE = N // NUM_TILES
FLUSH_CHUNK = 125
N_FLUSH = NODES_PER_TILE // FLUSH_CHUNK


def _seg_scratch():
    return [
        pltpu.VMEM((CHUNKS_PER_TILE, EDGE_CHUNK), jnp.int32),
        pltpu.VMEM((CHUNKS_PER_TILE, EDGE_CHUNK), jnp.int32),
        pltpu.VMEM((EDGE_CHUNK, 64), jnp.float32),
        pltpu.VMEM((EDGE_CHUNK, 64), jnp.float32),
        pltpu.VMEM((EDGE_CHUNK, 64), jnp.float32),
        pltpu.VMEM_SHARED((N, 64), jnp.float32),
        pltpu.SemaphoreType.DMA,
        pltpu.SemaphoreType.DMA,
    ]


def _seg_pass(c, s, tbl_pair, out_pair, zeros_h,
              idx_s, idx_d, rowsA, rowsB, rowsC, accum, gsem, ssem):
    nbase = s * NODES_PER_TILE
    pltpu.sync_copy(zeros_h, accum.at[pl.ds(nbase, NODES_PER_TILE)])
    plsc.subcore_barrier()

    t0, t1 = tbl_pair
    bufs = (rowsA, rowsB, rowsC)

    def gf(chunk, q):
        @pl.when(c == 0)
        def _():
            pltpu.async_copy(t0.at[idx_s.at[chunk]], bufs[q], gsem)

        @pl.when(c == 1)
        def _():
            pltpu.async_copy(t1.at[idx_s.at[chunk]], bufs[q], gsem)

    def gw(q):
        pltpu.make_async_copy(t0.at[idx_s.at[0]], bufs[q], gsem).wait()

    def sf(chunk, q):
        pltpu.async_copy(bufs[q], accum.at[idx_d.at[chunk]], ssem, add=True)

    def sw():
        pltpu.make_async_copy(bufs[0], accum.at[idx_d.at[0]], ssem).wait()

    gf(0, 0)
    gf(1, 1)
    gw(0); sf(0, 0); gf(2, 2)
    gw(1); sf(1, 1); sw(); gf(3, 0)
    gw(2); sf(2, 2); sw(); gf(4, 1)

    @pl.loop(0, (CHUNKS_PER_TILE - 5) // 3)
    def _(j):
        n = 3 + 3 * j
        gw(0); sf(n, 0); sw(); gf(n + 2, 2)
        gw(1); sf(n + 1, 1); sw(); gf(n + 3, 0)
        gw(2); sf(n + 2, 2); sw(); gf(n + 4, 1)

    gw(0); sf(CHUNKS_PER_TILE - 2, 0); sw()
    gw(1); sf(CHUNKS_PER_TILE - 1, 1); sw()
    sw()

    plsc.subcore_barrier()

    o0, o1 = out_pair

    @pl.when(c == 0)
    def _():
        pltpu.sync_copy(accum.at[pl.ds(nbase, NODES_PER_TILE)],
                        o0.at[pl.ds(nbase, NODES_PER_TILE)])

    @pl.when(c == 1)
    def _():
        pltpu.sync_copy(accum.at[pl.ds(nbase, NODES_PER_TILE)],
                        o1.at[pl.ds(nbase, NODES_PER_TILE)])

    plsc.subcore_barrier()


def _stage_indices(src_h, dst_h, s, idx_s, idx_d):
    pltpu.sync_copy(src_h.at[pl.ds(s * CHUNKS_PER_TILE, CHUNKS_PER_TILE)],
                    idx_s)
    pltpu.sync_copy(dst_h.at[pl.ds(s * CHUNKS_PER_TILE, CHUNKS_PER_TILE)],
                    idx_d)


def _seg_body_gc1(src_h, dst_h, zeros_h, t0, t1, out0, out1, *scr):
    c = lax.axis_index("c")
    s = lax.axis_index("s")
    _stage_indices(src_h, dst_h, s, scr[0], scr[1])
    _seg_pass(c, s, (t0, t1), (out0, out1), zeros_h, *scr)


def _seg_gc1(src2d, dst2d, zeros_h, t0, t1):
    mesh = plsc.VectorSubcoreMesh(core_axis_name="c", subcore_axis_name="s",
                                  num_cores=2, num_subcores=NUM_TILES)
    out = jax.ShapeDtypeStruct((N, 64), jnp.float32)
    f = pl.kernel(
        _seg_body_gc1,
        out_type=(out, out),
        mesh=mesh,
        compiler_params=pltpu.CompilerParams(use_tc_tiling_on_sc=False),
        scratch_types=_seg_scratch(),
    )
    return f(src2d, dst2d, zeros_h, t0, t1)


def _seg_body_gc2(src_h, dst_h, zeros_h, t0, t1, t2, t3,
                  out0, out1, out2, out3, *scr):
    c = lax.axis_index("c")
    s = lax.axis_index("s")
    _stage_indices(src_h, dst_h, s, scr[0], scr[1])
    _seg_pass(c, s, (t0, t2), (out0, out2), zeros_h, *scr)
    _seg_pass(c, s, (t1, t3), (out1, out3), zeros_h, *scr)


def _seg_gc2(src2d, dst2d, zeros_h, t0, t1, t2, t3):
    mesh = plsc.VectorSubcoreMesh(core_axis_name="c", subcore_axis_name="s",
                                  num_cores=2, num_subcores=NUM_TILES)
    out = jax.ShapeDtypeStruct((N, 64), jnp.float32)
    f = pl.kernel(
        _seg_body_gc2,
        out_type=(out, out, out, out),
        mesh=mesh,
        compiler_params=pltpu.CompilerParams(use_tc_tiling_on_sc=False),
        scratch_types=_seg_scratch(),
    )
    return f(src2d, dst2d, zeros_h, t0, t1, t2, t3)


ROWS_A = 1000


def _gc1_body(a0, a1, xb, wr, wx, b1, h1, g0, g1, g2, g3):
    agg = jnp.concatenate([a0[...], a1[...]], axis=1)
    h = jnp.dot(agg, wr[...], preferred_element_type=jnp.float32)
    h += jnp.dot(xb[...], wx[...], preferred_element_type=jnp.float32)
    h = jnp.maximum(h + b1[...], 0.0)
    h1[...] = h
    g0[...] = h[:, 0:64]
    g1[...] = h[:, 64:128]
    g2[...] = h[:, 128:192]
    g3[...] = h[:, 192:256]


def _gc1_dense(agg0, agg1, x, W_rel1, W_root1, b1):
    g_spec = pl.BlockSpec((ROWS_A, 64), lambda i: (i, 0))
    out64 = jax.ShapeDtypeStruct((N, 64), jnp.float32)
    return pl.pallas_call(
        _gc1_body,
        grid=(N // ROWS_A,),
        in_specs=[
            g_spec,
            g_spec,
            pl.BlockSpec((ROWS_A, IN), lambda i: (i, 0)),
            pl.BlockSpec((IN, HID), lambda i: (0, 0)),
            pl.BlockSpec((IN, HID), lambda i: (0, 0)),
            pl.BlockSpec((1, HID), lambda i: (0, 0)),
        ],
        out_specs=[pl.BlockSpec((ROWS_A, HID), lambda i: (i, 0)),
                   g_spec, g_spec, g_spec, g_spec],
        out_shape=[jax.ShapeDtypeStruct((N, HID), jnp.float32),
                   out64, out64, out64, out64],
    )(agg0, agg1, x, W_rel1, W_root1, b1)


NL_B = 40
FLAT = N_PER * HID


def _gc2_heads_body(a0, a1, a2, a3, h1b, wr, wx, b2, wl, wls, bl, bls,
                    loc_ref, ls_ref):
    i = pl.program_id(0)
    agg = jnp.concatenate([a0[...], a1[...], a2[...], a3[...]],
                          axis=2).reshape(BATCH * NL_B, HID)
    h = jnp.dot(agg, wr[...], preferred_element_type=jnp.float32)
    h += jnp.dot(h1b[...].reshape(BATCH * NL_B, HID), wx[...],
                 preferred_element_type=jnp.float32)
    h2 = jnp.maximum(h + b2[...], 0.0)
    flat = h2.reshape(BATCH, NL_B * HID)

    @pl.when(i == 0)
    def _():
        loc_ref[...] = jnp.broadcast_to(bl[...], (BATCH, LAT))
        ls_ref[...] = jnp.broadcast_to(bls[...], (BATCH, LAT))

    flat16 = flat.astype(jnp.bfloat16)
    loc_ref[...] += jnp.dot(
        flat16, wl[...].reshape(NL_B * HID, LAT).astype(jnp.bfloat16),
        preferred_element_type=jnp.float32)
    ls_ref[...] += jnp.dot(
        flat16, wls[...].reshape(NL_B * HID, LAT).astype(jnp.bfloat16),
        preferred_element_type=jnp.float32)


def _gc2_heads(agg2s, h1, W_rel2, W_root2, b2, W_loc, b_loc, W_ls, b_ls):
    a_spec = pl.BlockSpec((BATCH, NL_B, 64), lambda i: (0, i, 0))
    w_spec = pl.BlockSpec((NL_B, HID, LAT), lambda i: (i, 0, 0))
    out_spec = pl.BlockSpec((BATCH, LAT), lambda i: (0, 0))
    out = jax.ShapeDtypeStruct((BATCH, LAT), jnp.float32)
    a3d = [a.reshape(BATCH, N_PER, 64) for a in agg2s]
    return pl.pallas_call(
        _gc2_heads_body,
        grid=(N_PER // NL_B,),
        in_specs=[
            a_spec, a_spec, a_spec, a_spec,
            pl.BlockSpec((BATCH, NL_B, HID), lambda i: (0, i, 0)),
            pl.BlockSpec((HID, HID), lambda i: (0, 0)),
            pl.BlockSpec((HID, HID), lambda i: (0, 0)),
            pl.BlockSpec((1, HID), lambda i: (0, 0)),
            w_spec, w_spec,
            pl.BlockSpec((1, LAT), lambda i: (0, 0)),
            pl.BlockSpec((1, LAT), lambda i: (0, 0)),
        ],
        out_specs=[out_spec, out_spec],
        out_shape=[out, out],
        compiler_params=pltpu.CompilerParams(
            dimension_semantics=("arbitrary",)),
    )(*a3d, h1.reshape(BATCH, N_PER, HID), W_rel2, W_root2, b2,
      W_loc.reshape(N_PER, HID, LAT), W_ls.reshape(N_PER, HID, LAT),
      b_loc, b_ls)


@jax.jit
def kernel(x, edge_index, batch, W_rel1, b1, W_root1, W_rel2, b2, W_root2,
           W_loc, b_loc, W_ls, b_ls):
    src2d = edge_index[0].reshape(E // EDGE_CHUNK, EDGE_CHUNK)
    dst2d = edge_index[1].reshape(E // EDGE_CHUNK, EDGE_CHUNK)
    zeros_h = jnp.zeros((NODES_PER_TILE, 64), jnp.float32)
    x0 = x[:, 0:64]
    x1 = x[:, 64:128]

    agg1_0, agg1_1 = _seg_gc1(src2d, dst2d, zeros_h, x0, x1)
    h1, hg0, hg1, hg2, hg3 = _gc1_dense(
        agg1_0, agg1_1, x, W_rel1, W_root1, b1.reshape(1, HID))
    agg2s = _seg_gc2(src2d, dst2d, zeros_h, hg0, hg1, hg2, hg3)
    loc, ls = _gc2_heads(agg2s, h1, W_rel2, W_root2, b2.reshape(1, HID),
                         W_loc, b_loc.reshape(1, LAT), W_ls,
                         b_ls.reshape(1, LAT))
    return loc, ls


# speedup vs baseline: 1.0591x; 1.0361x over previous
import jax
import jax.numpy as jnp
from jax import lax
from jax.experimental import pallas as pl
from jax.experimental.pallas import tpu as pltpu
from jax.experimental.pallas import tpu_sc as plsc

N = 16000
E = 256000
IN = 128
HID = 256
LAT = 64
BATCH = 16
N_PER = 1000

NUM_TILES = 16
EDGE_CHUNK = 128
CHUNKS_PER_TILE = E // (NUM_TILES * EDGE_CHUNK)
NODES_PER_TILE = N // NUM_TILES
FLUSH_CHUNK = 125
N_FLUSH = NODES_PER_TILE // FLUSH_CHUNK


def _seg_scratch():
    return [
        pltpu.VMEM((CHUNKS_PER_TILE, EDGE_CHUNK), jnp.int32),
        pltpu.VMEM((CHUNKS_PER_TILE, EDGE_CHUNK), jnp.int32),
        pltpu.VMEM((EDGE_CHUNK, 64), jnp.float32),
        pltpu.VMEM((EDGE_CHUNK, 64), jnp.float32),
        pltpu.VMEM((EDGE_CHUNK, 64), jnp.float32),
        pltpu.VMEM((EDGE_CHUNK, 64), jnp.float32),
        pltpu.VMEM_SHARED((N, 64), jnp.float32),
        pltpu.SemaphoreType.DMA,
        pltpu.SemaphoreType.DMA,
    ]


def _seg_pass(c, s, tbl_pair, out_pair, zeros_h,
              idx_s, idx_d, rowsA, rowsB, rowsC, rowsD, accum, gsem, ssem):
    nbase = s * NODES_PER_TILE
    pltpu.sync_copy(zeros_h, accum.at[pl.ds(nbase, NODES_PER_TILE)])
    plsc.subcore_barrier()

    t0, t1 = tbl_pair
    bufs = (rowsA, rowsB, rowsC, rowsD)

    def gf(chunk, q):
        @pl.when(c == 0)
        def _():
            pltpu.async_copy(t0.at[idx_s.at[chunk]], bufs[q], gsem)

        @pl.when(c == 1)
        def _():
            pltpu.async_copy(t1.at[idx_s.at[chunk]], bufs[q], gsem)

    def gw(q):
        pltpu.make_async_copy(t0.at[idx_s.at[0]], bufs[q], gsem).wait()

    def sf(chunk, q):
        pltpu.async_copy(bufs[q], accum.at[idx_d.at[chunk]], ssem, add=True)

    def sw():
        pltpu.make_async_copy(bufs[0], accum.at[idx_d.at[0]], ssem).wait()

    gf(0, 0)
    gf(1, 1)
    gf(2, 2)
    gw(0); sf(0, 0); gf(3, 3)
    gw(1); sf(1, 1); sw(); gf(4, 0)
    gw(2); sf(2, 2); sw(); gf(5, 1)
    gw(3); sf(3, 3); sw(); gf(6, 2)

    @pl.loop(0, (CHUNKS_PER_TILE - 9) // 4)
    def _(j):
        n = 4 + 4 * j
        gw(0); sf(n, 0); sw(); gf(n + 3, 3)
        gw(1); sf(n + 1, 1); sw(); gf(n + 4, 0)
        gw(2); sf(n + 2, 2); sw(); gf(n + 5, 1)
        gw(3); sf(n + 3, 3); sw(); gf(n + 6, 2)

    gw(0); sf(CHUNKS_PER_TILE - 5, 0); sw(); gf(CHUNKS_PER_TILE - 2, 3)
    gw(1); sf(CHUNKS_PER_TILE - 4, 1); sw(); gf(CHUNKS_PER_TILE - 1, 0)
    gw(2); sf(CHUNKS_PER_TILE - 3, 2); sw()
    gw(3); sf(CHUNKS_PER_TILE - 2, 3); sw()
    gw(0); sf(CHUNKS_PER_TILE - 1, 0); sw()
    sw()

    plsc.subcore_barrier()

    o0, o1 = out_pair

    @pl.when(c == 0)
    def _():
        pltpu.sync_copy(accum.at[pl.ds(nbase, NODES_PER_TILE)],
                        o0.at[pl.ds(nbase, NODES_PER_TILE)])

    @pl.when(c == 1)
    def _():
        pltpu.sync_copy(accum.at[pl.ds(nbase, NODES_PER_TILE)],
                        o1.at[pl.ds(nbase, NODES_PER_TILE)])

    plsc.subcore_barrier()


def _stage_indices(src_h, dst_h, s, idx_s, idx_d):
    pltpu.sync_copy(src_h.at[pl.ds(s * CHUNKS_PER_TILE, CHUNKS_PER_TILE)],
                    idx_s)
    pltpu.sync_copy(dst_h.at[pl.ds(s * CHUNKS_PER_TILE, CHUNKS_PER_TILE)],
                    idx_d)


def _seg_body_gc1(src_h, dst_h, zeros_h, t0, t1, out0, out1, *scr):
    c = lax.axis_index("c")
    s = lax.axis_index("s")
    _stage_indices(src_h, dst_h, s, scr[0], scr[1])
    _seg_pass(c, s, (t0, t1), (out0, out1), zeros_h, *scr)


def _seg_gc1(src2d, dst2d, zeros_h, t0, t1):
    mesh = plsc.VectorSubcoreMesh(core_axis_name="c", subcore_axis_name="s",
                                  num_cores=2, num_subcores=NUM_TILES)
    out = jax.ShapeDtypeStruct((N, 64), jnp.float32)
    f = pl.kernel(
        _seg_body_gc1,
        out_type=(out, out),
        mesh=mesh,
        compiler_params=pltpu.CompilerParams(use_tc_tiling_on_sc=False),
        scratch_types=_seg_scratch(),
    )
    return f(src2d, dst2d, zeros_h, t0, t1)


def _seg_body_gc2(src_h, dst_h, zeros_h, t0, t1, t2, t3,
                  out0, out1, out2, out3, *scr):
    c = lax.axis_index("c")
    s = lax.axis_index("s")
    _stage_indices(src_h, dst_h, s, scr[0], scr[1])
    _seg_pass(c, s, (t0, t2), (out0, out2), zeros_h, *scr)
    _seg_pass(c, s, (t1, t3), (out1, out3), zeros_h, *scr)


def _seg_gc2(src2d, dst2d, zeros_h, t0, t1, t2, t3):
    mesh = plsc.VectorSubcoreMesh(core_axis_name="c", subcore_axis_name="s",
                                  num_cores=2, num_subcores=NUM_TILES)
    out = jax.ShapeDtypeStruct((N, 64), jnp.float32)
    f = pl.kernel(
        _seg_body_gc2,
        out_type=(out, out, out, out),
        mesh=mesh,
        compiler_params=pltpu.CompilerParams(use_tc_tiling_on_sc=False),
        scratch_types=_seg_scratch(),
    )
    return f(src2d, dst2d, zeros_h, t0, t1, t2, t3)


ROWS_A = 1000


def _gc1_body(a0, a1, xb, wr, wx, b1, h1, g0, g1, g2, g3):
    agg = jnp.concatenate([a0[...], a1[...]], axis=1)
    h = jnp.dot(agg, wr[...], preferred_element_type=jnp.float32)
    h += jnp.dot(xb[...], wx[...], preferred_element_type=jnp.float32)
    h = jnp.maximum(h + b1[...], 0.0)
    h1[...] = h
    g0[...] = h[:, 0:64]
    g1[...] = h[:, 64:128]
    g2[...] = h[:, 128:192]
    g3[...] = h[:, 192:256]


def _gc1_dense(agg0, agg1, x, W_rel1, W_root1, b1):
    g_spec = pl.BlockSpec((ROWS_A, 64), lambda i: (i, 0))
    out64 = jax.ShapeDtypeStruct((N, 64), jnp.float32)
    return pl.pallas_call(
        _gc1_body,
        grid=(N // ROWS_A,),
        in_specs=[
            g_spec,
            g_spec,
            pl.BlockSpec((ROWS_A, IN), lambda i: (i, 0)),
            pl.BlockSpec((IN, HID), lambda i: (0, 0)),
            pl.BlockSpec((IN, HID), lambda i: (0, 0)),
            pl.BlockSpec((1, HID), lambda i: (0, 0)),
        ],
        out_specs=[pl.BlockSpec((ROWS_A, HID), lambda i: (i, 0)),
                   g_spec, g_spec, g_spec, g_spec],
        out_shape=[jax.ShapeDtypeStruct((N, HID), jnp.float32),
                   out64, out64, out64, out64],
    )(agg0, agg1, x, W_rel1, W_root1, b1)


NL_B = 40
FLAT = N_PER * HID


def _gc2_heads_body(a0, a1, a2, a3, h1b, wr, wx, b2, wl, wls, bl, bls,
                    loc_ref, ls_ref):
    i = pl.program_id(0)
    agg = jnp.concatenate([a0[...], a1[...], a2[...], a3[...]],
                          axis=2).reshape(BATCH * NL_B, HID)
    h = jnp.dot(agg, wr[...], preferred_element_type=jnp.float32)
    h += jnp.dot(h1b[...].reshape(BATCH * NL_B, HID), wx[...],
                 preferred_element_type=jnp.float32)
    h2 = jnp.maximum(h + b2[...], 0.0)
    flat = h2.reshape(BATCH, NL_B * HID)

    @pl.when(i == 0)
    def _():
        loc_ref[...] = jnp.broadcast_to(bl[...], (BATCH, LAT))
        ls_ref[...] = jnp.broadcast_to(bls[...], (BATCH, LAT))

    loc_ref[...] += jnp.dot(flat, wl[...].reshape(NL_B * HID, LAT),
                            preferred_element_type=jnp.float32)
    ls_ref[...] += jnp.dot(flat, wls[...].reshape(NL_B * HID, LAT),
                           preferred_element_type=jnp.float32)


def _gc2_heads(agg2s, h1, W_rel2, W_root2, b2, W_loc, b_loc, W_ls, b_ls):
    a_spec = pl.BlockSpec((BATCH, NL_B, 64), lambda i: (0, i, 0))
    w_spec = pl.BlockSpec((NL_B, HID, LAT), lambda i: (i, 0, 0))
    out_spec = pl.BlockSpec((BATCH, LAT), lambda i: (0, 0))
    out = jax.ShapeDtypeStruct((BATCH, LAT), jnp.float32)
    a3d = [a.reshape(BATCH, N_PER, 64) for a in agg2s]
    return pl.pallas_call(
        _gc2_heads_body,
        grid=(N_PER // NL_B,),
        in_specs=[
            a_spec, a_spec, a_spec, a_spec,
            pl.BlockSpec((BATCH, NL_B, HID), lambda i: (0, i, 0)),
            pl.BlockSpec((HID, HID), lambda i: (0, 0)),
            pl.BlockSpec((HID, HID), lambda i: (0, 0)),
            pl.BlockSpec((1, HID), lambda i: (0, 0)),
            w_spec, w_spec,
            pl.BlockSpec((1, LAT), lambda i: (0, 0)),
            pl.BlockSpec((1, LAT), lambda i: (0, 0)),
        ],
        out_specs=[out_spec, out_spec],
        out_shape=[out, out],
        compiler_params=pltpu.CompilerParams(
            dimension_semantics=("arbitrary",)),
    )(*a3d, h1.reshape(BATCH, N_PER, HID), W_rel2, W_root2, b2,
      W_loc.reshape(N_PER, HID, LAT), W_ls.reshape(N_PER, HID, LAT),
      b_loc, b_ls)


@jax.jit
def kernel(x, edge_index, batch, W_rel1, b1, W_root1, W_rel2, b2, W_root2,
           W_loc, b_loc, W_ls, b_ls):
    src2d = edge_index[0].reshape(E // EDGE_CHUNK, EDGE_CHUNK)
    dst2d = edge_index[1].reshape(E // EDGE_CHUNK, EDGE_CHUNK)
    zeros_h = jnp.zeros((NODES_PER_TILE, 64), jnp.float32)
    x0 = x[:, 0:64]
    x1 = x[:, 64:128]

    agg1_0, agg1_1 = _seg_gc1(src2d, dst2d, zeros_h, x0, x1)
    h1, hg0, hg1, hg2, hg3 = _gc1_dense(
        agg1_0, agg1_1, x, W_rel1, W_root1, b1.reshape(1, HID))
    agg2s = _seg_gc2(src2d, dst2d, zeros_h, hg0, hg1, hg2, hg3)
    loc, ls = _gc2_heads(agg2s, h1, W_rel2, W_root2, b2.reshape(1, HID),
                         W_loc, b_loc.reshape(1, LAT), W_ls,
                         b_ls.reshape(1, LAT))
    return loc, ls
